# baseline (speedup 1.0000x reference)
import functools

import jax
import jax.numpy as jnp
from jax import lax
from jax.experimental import pallas as pl
from jax.experimental.pallas import tpu as pltpu
from jax.experimental.pallas import tpu_sc as plsc

_ROWS = 16384
_COLS = 512
_WPAD = 9040
_NW = 32
_TROWS = _ROWS // _NW
_CHUNK = 32
_NCHUNK = _TROWS // _CHUNK
_NBUF = 6
_LOOKAHEAD = 3


def _make_sc_scale():
    mesh = plsc.VectorSubcoreMesh(core_axis_name="c", subcore_axis_name="s")

    @functools.partial(
        pl.kernel,
        mesh=mesh,
        out_type=jax.ShapeDtypeStruct((_ROWS, _COLS), jnp.float32),
        scratch_types=[
            pltpu.VMEM((16,), jnp.int32),
            pltpu.VMEM((16,), jnp.float32),
            pltpu.VMEM((_NBUF, _CHUNK, _COLS), jnp.float32),
            pltpu.SemaphoreType.DMA,
            pltpu.SemaphoreType.DMA((_NBUF,)),
            pltpu.SemaphoreType.DMA((_NBUF,)),
        ],
    )
    def _scale(x_hbm, w_hbm, idx_hbm, out_hbm, idx_v, vals_v, bufs, gsem,
               in_sems, out_sems):
        cid = lax.axis_index("c")
        sid = lax.axis_index("s")
        wid = sid * 2 + cid
        base = wid * _TROWS

        pltpu.sync_copy(idx_hbm, idx_v)
        pltpu.async_copy(w_hbm.at[idx_v], vals_v, gsem).wait()
        s16 = 1.0 / (1.0 + jnp.exp(-vals_v[...]))

        def in_copy(c):
            return pltpu.make_async_copy(
                x_hbm.at[pl.ds(base + c * _CHUNK, _CHUNK), :],
                bufs.at[c % _NBUF],
                in_sems.at[c % _NBUF],
            )

        def out_copy(c):
            return pltpu.make_async_copy(
                bufs.at[c % _NBUF],
                out_hbm.at[pl.ds(base + c * _CHUNK, _CHUNK), :],
                out_sems.at[c % _NBUF],
            )

        for c in range(_LOOKAHEAD):
            in_copy(c).start()

        for c in range(_NCHUNK):
            nxt = c + _LOOKAHEAD
            if nxt < _NCHUNK:
                prev = nxt - _NBUF
                if prev >= 0:
                    out_copy(prev).wait()
                in_copy(nxt).start()
            in_copy(c).wait()
            buf = bufs.at[c % _NBUF]

            def body(r, _):
                for k in range(_COLS // 16):
                    buf[r, 16 * k:16 * (k + 1)] = (
                        buf[r, 16 * k:16 * (k + 1)] * s16)
                return 0

            lax.fori_loop(0, _CHUNK, body, 0)
            out_copy(c).start()

        for c in range(max(0, _NCHUNK - _NBUF), _NCHUNK):
            out_copy(c).wait()

    return _scale


_sc_scale = _make_sc_scale()


def kernel(input_features, part_cls, obj_cls, W):
    p = jnp.asarray(part_cls, jnp.int32)
    o = jnp.asarray(obj_cls, jnp.int32)
    idx16 = jnp.full((16,), p * 95 + o, dtype=jnp.int32)
    w_flat = jnp.pad(W.reshape(-1), (0, _WPAD - 95 * 95))
    return _sc_scale(input_features, w_flat, idx16)
